# Initial kernel scaffold; baseline (speedup 1.0000x reference)
import jax
import jax.numpy as jnp
from jax.experimental import pallas as pl


def kernel(x, edge_index, Ws, bs):
    raise NotImplementedError("write your pallas kernel here")


# speedup vs baseline: 13.7505x; 13.7505x over previous
import functools

import jax
import jax.numpy as jnp
from jax import lax
from jax.experimental import pallas as pl
from jax.experimental.pallas import tpu as pltpu
from jax.experimental.pallas import tpu_sc as plsc

N_NODES = 10000
N_PAD = 10240
DIM = 128
NUM_EDGES = 320000
NUM_LAYERS = 10

NUM_CORES = 2
NUM_SUBCORES = 16
NW = NUM_CORES * NUM_SUBCORES
EDGES_PER_W = NUM_EDGES // NW
CHUNK = 125
CHUNKS = EDGES_PER_W // CHUNK
ROWS_PER_TILE = N_PAD // NUM_SUBCORES

_MESH = plsc.VectorSubcoreMesh(core_axis_name="c", subcore_axis_name="s")


@functools.partial(
    pl.kernel,
    mesh=_MESH,
    out_type=jax.ShapeDtypeStruct((NUM_CORES, N_PAD, DIM), jnp.float32),
    scratch_types=[
        pltpu.VMEM((CHUNKS, CHUNK), jnp.int32),
        pltpu.VMEM((CHUNKS, CHUNK), jnp.int32),
        pltpu.VMEM((CHUNK, DIM), jnp.float32),
        pltpu.VMEM_SHARED((N_PAD, DIM), jnp.float32),
        pltpu.SemaphoreType.DMA,
    ],
)
def _sc_aggregate(g_hbm, src_hbm, dst_hbm, zero_hbm, out_hbm,
                  sidx, didx, rows, accum, sem):
    cid = lax.axis_index("c")
    sid = lax.axis_index("s")
    wid = sid * NUM_CORES + cid
    base = sid * ROWS_PER_TILE

    pltpu.sync_copy(src_hbm.at[wid], sidx)
    pltpu.sync_copy(dst_hbm.at[wid], didx)
    pltpu.sync_copy(zero_hbm.at[pl.ds(base, ROWS_PER_TILE)],
                    accum.at[pl.ds(base, ROWS_PER_TILE)])
    plsc.subcore_barrier()

    def body(j, carry):
        pltpu.async_copy(g_hbm.at[sidx.at[j]], rows, sem).wait()
        pltpu.sync_copy(rows, accum.at[didx.at[j]], add=True)
        return carry

    lax.fori_loop(0, CHUNKS, body, 0)

    plsc.subcore_barrier()
    pltpu.sync_copy(accum.at[pl.ds(base, ROWS_PER_TILE)],
                    out_hbm.at[cid, pl.ds(base, ROWS_PER_TILE)])


@functools.partial(
    pl.kernel,
    mesh=_MESH,
    out_type=jax.ShapeDtypeStruct((NUM_CORES, N_PAD, DIM), jnp.float32),
    scratch_types=[
        pltpu.VMEM((CHUNKS, CHUNK), jnp.int32),
        pltpu.VMEM((CHUNK, DIM), jnp.float32),
        pltpu.VMEM_SHARED((N_PAD, DIM), jnp.float32),
    ],
)
def _sc_degree(dst_hbm, ones_hbm, zero_hbm, out_hbm, didx, ones_v, accum):
    cid = lax.axis_index("c")
    sid = lax.axis_index("s")
    wid = sid * NUM_CORES + cid
    base = sid * ROWS_PER_TILE

    pltpu.sync_copy(dst_hbm.at[wid], didx)
    pltpu.sync_copy(ones_hbm, ones_v)
    pltpu.sync_copy(zero_hbm.at[pl.ds(base, ROWS_PER_TILE)],
                    accum.at[pl.ds(base, ROWS_PER_TILE)])
    plsc.subcore_barrier()

    def body(j, carry):
        pltpu.sync_copy(ones_v, accum.at[didx.at[j]], add=True)
        return carry

    lax.fori_loop(0, CHUNKS, body, 0)

    plsc.subcore_barrier()
    pltpu.sync_copy(accum.at[pl.ds(base, ROWS_PER_TILE)],
                    out_hbm.at[cid, pl.ds(base, ROWS_PER_TILE)])


_ROWS_BLK = 1024


def _tc_pre(degcols, x, w0):
    def body(deg_ref, x_ref, w_ref, dis_ref, g_ref):
        deg = deg_ref[0, :, 0:1] + deg_ref[1, :, 0:1] + 1.0
        dis = lax.rsqrt(deg)
        dis_ref[...] = dis
        g_ref[...] = dis * jnp.dot(x_ref[...], w_ref[...],
                                   preferred_element_type=jnp.float32,
                                   precision=lax.Precision.DEFAULT)

    return pl.pallas_call(
        body,
        grid=(N_PAD // _ROWS_BLK,),
        in_specs=[
            pl.BlockSpec((NUM_CORES, _ROWS_BLK, DIM), lambda i: (0, i, 0)),
            pl.BlockSpec((_ROWS_BLK, DIM), lambda i: (i, 0)),
            pl.BlockSpec((DIM, DIM), lambda i: (0, 0)),
        ],
        out_specs=[
            pl.BlockSpec((_ROWS_BLK, 1), lambda i: (i, 0)),
            pl.BlockSpec((_ROWS_BLK, DIM), lambda i: (i, 0)),
        ],
        out_shape=[
            jax.ShapeDtypeStruct((N_PAD, 1), jnp.float32),
            jax.ShapeDtypeStruct((N_PAD, DIM), jnp.float32),
        ],
    )(degcols, x, w0)


def _tc_mid(s, g, dis, b, w_next):
    def body(s_ref, g_ref, dis_ref, b_ref, w_ref, out_ref):
        t = dis_ref[...] * (s_ref[0] + s_ref[1] + g_ref[...]) + b_ref[...]
        h = jnp.maximum(t, 0.0)
        out_ref[...] = dis_ref[...] * jnp.dot(h, w_ref[...],
                                              preferred_element_type=jnp.float32,
                                              precision=lax.Precision.DEFAULT)

    return pl.pallas_call(
        body,
        grid=(N_PAD // _ROWS_BLK,),
        in_specs=[
            pl.BlockSpec((NUM_CORES, _ROWS_BLK, DIM), lambda i: (0, i, 0)),
            pl.BlockSpec((_ROWS_BLK, DIM), lambda i: (i, 0)),
            pl.BlockSpec((_ROWS_BLK, 1), lambda i: (i, 0)),
            pl.BlockSpec((1, DIM), lambda i: (0, 0)),
            pl.BlockSpec((DIM, DIM), lambda i: (0, 0)),
        ],
        out_specs=pl.BlockSpec((_ROWS_BLK, DIM), lambda i: (i, 0)),
        out_shape=jax.ShapeDtypeStruct((N_PAD, DIM), jnp.float32),
    )(s, g, dis, b, w_next)


def _tc_post(s, g, dis, b):
    def body(s_ref, g_ref, dis_ref, b_ref, out_ref):
        out_ref[...] = (dis_ref[...] * (s_ref[0] + s_ref[1] + g_ref[...])
                        + b_ref[...])

    return pl.pallas_call(
        body,
        grid=(N_PAD // _ROWS_BLK,),
        in_specs=[
            pl.BlockSpec((NUM_CORES, _ROWS_BLK, DIM), lambda i: (0, i, 0)),
            pl.BlockSpec((_ROWS_BLK, DIM), lambda i: (i, 0)),
            pl.BlockSpec((_ROWS_BLK, 1), lambda i: (i, 0)),
            pl.BlockSpec((1, DIM), lambda i: (0, 0)),
        ],
        out_specs=pl.BlockSpec((_ROWS_BLK, DIM), lambda i: (i, 0)),
        out_shape=jax.ShapeDtypeStruct((N_PAD, DIM), jnp.float32),
    )(s, g, dis, b)


def kernel(x, edge_index, Ws, bs):
    src = edge_index[0].astype(jnp.int32).reshape(NW, CHUNKS, CHUNK)
    dst = edge_index[1].astype(jnp.int32).reshape(NW, CHUNKS, CHUNK)
    ones_c = jnp.ones((CHUNK, DIM), jnp.float32)
    zeros_d = jnp.zeros((N_PAD, DIM), jnp.float32)
    x_pad = jnp.pad(x, ((0, N_PAD - N_NODES), (0, 0)))

    degcols = _sc_degree(dst, ones_c, zeros_d)
    dis, g = _tc_pre(degcols, x_pad, Ws[0])
    out = None
    for l in range(NUM_LAYERS):
        s = _sc_aggregate(g, src, dst, zeros_d)
        if l < NUM_LAYERS - 1:
            g = _tc_mid(s, g, dis, bs[l].reshape(1, DIM), Ws[l + 1])
        else:
            out = _tc_post(s, g, dis, bs[l].reshape(1, DIM))
    return out[:N_NODES]


# speedup vs baseline: 16.7324x; 1.2169x over previous
import functools

import jax
import jax.numpy as jnp
from jax import lax
from jax.experimental import pallas as pl
from jax.experimental.pallas import tpu as pltpu
from jax.experimental.pallas import tpu_sc as plsc

N_NODES = 10000
N_PAD = 10240
DIM = 128
NUM_EDGES = 320000
NUM_LAYERS = 10

NUM_CORES = 2
NUM_SUBCORES = 16
NW = NUM_CORES * NUM_SUBCORES
EDGES_PER_W = NUM_EDGES // NW
CHUNK = 125
CHUNKS = EDGES_PER_W // CHUNK
ROWS_PER_TILE = N_PAD // NUM_SUBCORES
PHASES = 2
HC = CHUNKS // PHASES

_MESH = plsc.VectorSubcoreMesh(core_axis_name="c", subcore_axis_name="s")


@functools.partial(
    pl.kernel,
    mesh=_MESH,
    out_type=jax.ShapeDtypeStruct((NUM_CORES, N_PAD, DIM), jnp.float32),
    scratch_types=[
        pltpu.VMEM((HC, CHUNK), jnp.int32),
        pltpu.VMEM((HC, CHUNK), jnp.int32),
        pltpu.VMEM((2, CHUNK, DIM), jnp.float32),
        pltpu.VMEM_SHARED((N_PAD, DIM), jnp.float32),
        pltpu.SemaphoreType.DMA,
        pltpu.SemaphoreType.DMA,
    ],
)
def _sc_aggregate(g_hbm, src_hbm, dst_hbm, zero_hbm, out_hbm,
                  sidx, didx, rows, accum, gsem, ssem):
    cid = lax.axis_index("c")
    sid = lax.axis_index("s")
    wid = sid * NUM_CORES + cid
    base = sid * ROWS_PER_TILE

    pltpu.sync_copy(zero_hbm.at[pl.ds(base, ROWS_PER_TILE)],
                    accum.at[pl.ds(base, ROWS_PER_TILE)])
    plsc.subcore_barrier()

    for p in range(PHASES):
        pltpu.sync_copy(src_hbm.at[wid, pl.ds(p * HC, HC)], sidx)
        pltpu.sync_copy(dst_hbm.at[wid, pl.ds(p * HC, HC)], didx)

        pltpu.async_copy(g_hbm.at[sidx.at[0]], rows.at[0], gsem)

        def body(q, carry):
            b = lax.rem(q, 2)
            bn = lax.rem(q + 1, 2)
            pltpu.make_async_copy(g_hbm.at[sidx.at[q]], rows.at[b],
                                  gsem).wait()

            @pl.when(q > 0)
            def _():
                pltpu.make_async_copy(rows.at[bn], accum.at[didx.at[q - 1]],
                                      ssem).wait()

            pltpu.async_copy(g_hbm.at[sidx.at[lax.rem(q + 1, HC)]],
                             rows.at[bn], gsem)
            pltpu.async_copy(rows.at[b], accum.at[didx.at[q]], ssem, add=True)
            return carry

        lax.fori_loop(0, HC, body, 0)

        pltpu.make_async_copy(rows.at[lax.rem(HC - 1, 2)],
                              accum.at[didx.at[HC - 1]], ssem).wait()
        pltpu.make_async_copy(g_hbm.at[sidx.at[0]], rows.at[0], gsem).wait()

    plsc.subcore_barrier()
    pltpu.sync_copy(accum.at[pl.ds(base, ROWS_PER_TILE)],
                    out_hbm.at[cid, pl.ds(base, ROWS_PER_TILE)])


_ROWS_BLK = 1024


def _tc_pre(degcols, x, w0):
    def body(deg_ref, x_ref, w_ref, dis_ref, g_ref):
        deg = deg_ref[0, :, 0:1] + deg_ref[1, :, 0:1] + 1.0
        dis = lax.rsqrt(deg)
        dis_ref[...] = dis
        g_ref[...] = dis * jnp.dot(x_ref[...], w_ref[...],
                                   preferred_element_type=jnp.float32,
                                   precision=lax.Precision.DEFAULT)

    return pl.pallas_call(
        body,
        grid=(N_PAD // _ROWS_BLK,),
        in_specs=[
            pl.BlockSpec((NUM_CORES, _ROWS_BLK, DIM), lambda i: (0, i, 0)),
            pl.BlockSpec((_ROWS_BLK, DIM), lambda i: (i, 0)),
            pl.BlockSpec((DIM, DIM), lambda i: (0, 0)),
        ],
        out_specs=[
            pl.BlockSpec((_ROWS_BLK, 1), lambda i: (i, 0)),
            pl.BlockSpec((_ROWS_BLK, DIM), lambda i: (i, 0)),
        ],
        out_shape=[
            jax.ShapeDtypeStruct((N_PAD, 1), jnp.float32),
            jax.ShapeDtypeStruct((N_PAD, DIM), jnp.float32),
        ],
    )(degcols, x, w0)


def _tc_mid(s, g, dis, b, w_next):
    def body(s_ref, g_ref, dis_ref, b_ref, w_ref, out_ref):
        t = dis_ref[...] * (s_ref[0] + s_ref[1] + g_ref[...]) + b_ref[...]
        h = jnp.maximum(t, 0.0)
        out_ref[...] = dis_ref[...] * jnp.dot(h, w_ref[...],
                                              preferred_element_type=jnp.float32,
                                              precision=lax.Precision.DEFAULT)

    return pl.pallas_call(
        body,
        grid=(N_PAD // _ROWS_BLK,),
        in_specs=[
            pl.BlockSpec((NUM_CORES, _ROWS_BLK, DIM), lambda i: (0, i, 0)),
            pl.BlockSpec((_ROWS_BLK, DIM), lambda i: (i, 0)),
            pl.BlockSpec((_ROWS_BLK, 1), lambda i: (i, 0)),
            pl.BlockSpec((1, DIM), lambda i: (0, 0)),
            pl.BlockSpec((DIM, DIM), lambda i: (0, 0)),
        ],
        out_specs=pl.BlockSpec((_ROWS_BLK, DIM), lambda i: (i, 0)),
        out_shape=jax.ShapeDtypeStruct((N_PAD, DIM), jnp.float32),
    )(s, g, dis, b, w_next)


def _tc_post(s, g, dis, b):
    def body(s_ref, g_ref, dis_ref, b_ref, out_ref):
        out_ref[...] = (dis_ref[...] * (s_ref[0] + s_ref[1] + g_ref[...])
                        + b_ref[...])

    return pl.pallas_call(
        body,
        grid=(N_PAD // _ROWS_BLK,),
        in_specs=[
            pl.BlockSpec((NUM_CORES, _ROWS_BLK, DIM), lambda i: (0, i, 0)),
            pl.BlockSpec((_ROWS_BLK, DIM), lambda i: (i, 0)),
            pl.BlockSpec((_ROWS_BLK, 1), lambda i: (i, 0)),
            pl.BlockSpec((1, DIM), lambda i: (0, 0)),
        ],
        out_specs=pl.BlockSpec((_ROWS_BLK, DIM), lambda i: (i, 0)),
        out_shape=jax.ShapeDtypeStruct((N_PAD, DIM), jnp.float32),
    )(s, g, dis, b)


def kernel(x, edge_index, Ws, bs):
    src = edge_index[0].astype(jnp.int32).reshape(NW, CHUNKS, CHUNK)
    dst = edge_index[1].astype(jnp.int32).reshape(NW, CHUNKS, CHUNK)
    zeros_d = jnp.zeros((N_PAD, DIM), jnp.float32)
    ones_d = jnp.ones((N_PAD, DIM), jnp.float32)
    x_pad = jnp.pad(x, ((0, N_PAD - N_NODES), (0, 0)))

    degcols = _sc_aggregate(ones_d, src, dst, zeros_d)
    dis, g = _tc_pre(degcols, x_pad, Ws[0])
    out = None
    for l in range(NUM_LAYERS):
        s = _sc_aggregate(g, src, dst, zeros_d)
        if l < NUM_LAYERS - 1:
            g = _tc_mid(s, g, dis, bs[l].reshape(1, DIM), Ws[l + 1])
        else:
            out = _tc_post(s, g, dis, bs[l].reshape(1, DIM))
    return out[:N_NODES]


# speedup vs baseline: 22.6578x; 1.3541x over previous
import functools

import jax
import jax.numpy as jnp
from jax import lax
from jax.experimental import pallas as pl
from jax.experimental.pallas import tpu as pltpu
from jax.experimental.pallas import tpu_sc as plsc

N_NODES = 10000
N_PAD = 10240
DIM = 128
NUM_EDGES = 320000
NUM_LAYERS = 10

NUM_CORES = 2
NUM_SUBCORES = 16
NW = NUM_CORES * NUM_SUBCORES
EDGES_PER_W = NUM_EDGES // NW
CHUNK = 100
CHUNKS = EDGES_PER_W // CHUNK
ROWS_PER_TILE = N_PAD // NUM_SUBCORES
PHASES = 4
HC = CHUNKS // PHASES

_MESH = plsc.VectorSubcoreMesh(core_axis_name="c", subcore_axis_name="s")


@functools.partial(
    pl.kernel,
    mesh=_MESH,
    out_type=jax.ShapeDtypeStruct((NUM_CORES, N_PAD, DIM), jnp.float32),
    scratch_types=[
        pltpu.VMEM((HC, CHUNK), jnp.int32),
        pltpu.VMEM((HC, CHUNK), jnp.int32),
        pltpu.VMEM((3, CHUNK, DIM), jnp.float32),
        pltpu.VMEM_SHARED((N_PAD, DIM), jnp.float32),
        pltpu.SemaphoreType.DMA,
        pltpu.SemaphoreType.DMA,
        pltpu.SemaphoreType.DMA,
    ],
)
def _sc_aggregate(g_hbm, src_hbm, dst_hbm, zero_hbm, out_hbm,
                  sidx, didx, rows, accum, gsem0, gsem1, ssem):
    cid = lax.axis_index("c")
    sid = lax.axis_index("s")
    wid = sid * NUM_CORES + cid
    base = sid * ROWS_PER_TILE

    @pl.when(cid == 0)
    def _():
        pltpu.sync_copy(g_hbm.at[pl.ds(base, ROWS_PER_TILE)],
                        accum.at[pl.ds(base, ROWS_PER_TILE)])

    @pl.when(cid == 1)
    def _():
        pltpu.sync_copy(zero_hbm.at[pl.ds(base, ROWS_PER_TILE)],
                        accum.at[pl.ds(base, ROWS_PER_TILE)])

    plsc.subcore_barrier()

    for p in range(PHASES):
        pltpu.sync_copy(src_hbm.at[wid, p], sidx)
        pltpu.sync_copy(dst_hbm.at[wid, p], didx)

        pltpu.async_copy(g_hbm.at[sidx.at[0]], rows.at[0], gsem0)
        pltpu.async_copy(g_hbm.at[sidx.at[1]], rows.at[1], gsem1)

        def body(q, carry):
            b = lax.rem(q, 3)
            gpar = lax.rem(q, 2)

            @pl.when(gpar == 0)
            def _():
                pltpu.make_async_copy(g_hbm.at[sidx.at[q]], rows.at[b],
                                      gsem0).wait()

            @pl.when(gpar == 1)
            def _():
                pltpu.make_async_copy(g_hbm.at[sidx.at[q]], rows.at[b],
                                      gsem1).wait()

            @pl.when(q > 0)
            def _():
                pltpu.make_async_copy(rows.at[lax.rem(q + 2, 3)],
                                      accum.at[didx.at[q - 1]], ssem).wait()

            qn = lax.rem(q + 2, HC)
            bn = lax.rem(q + 2, 3)

            @pl.when(gpar == 0)
            def _():
                pltpu.async_copy(g_hbm.at[sidx.at[qn]], rows.at[bn], gsem0)

            @pl.when(gpar == 1)
            def _():
                pltpu.async_copy(g_hbm.at[sidx.at[qn]], rows.at[bn], gsem1)

            pltpu.async_copy(rows.at[b], accum.at[didx.at[q]], ssem, add=True)
            return carry

        lax.fori_loop(0, HC, body, 0)

        pltpu.make_async_copy(rows.at[lax.rem(HC - 1, 3)],
                              accum.at[didx.at[HC - 1]], ssem).wait()
        pltpu.make_async_copy(g_hbm.at[sidx.at[0]], rows.at[0], gsem0).wait()
        pltpu.make_async_copy(g_hbm.at[sidx.at[1]], rows.at[1], gsem1).wait()

    plsc.subcore_barrier()
    pltpu.sync_copy(accum.at[pl.ds(base, ROWS_PER_TILE)],
                    out_hbm.at[cid, pl.ds(base, ROWS_PER_TILE)])


@functools.partial(
    pl.kernel,
    mesh=_MESH,
    out_type=jax.ShapeDtypeStruct((NW, N_PAD // DIM, DIM), jnp.float32),
    scratch_types=[
        pltpu.VMEM((N_PAD // DIM, DIM), jnp.int32),
        pltpu.VMEM((N_PAD // DIM, DIM), jnp.float32),
    ],
    compiler_params=pltpu.CompilerParams(needs_layout_passes=False),
)
def _sc_degree(dst_hbm, out_hbm, didx, local):
    cid = lax.axis_index("c")
    sid = lax.axis_index("s")
    wid = sid * NUM_CORES + cid

    pltpu.sync_copy(dst_hbm.at[wid], didx)

    def zero(k, carry):
        local[k >> 3, pl.ds((k & 7) * 16, 16)] = jnp.zeros((16,), jnp.float32)
        return carry

    lax.fori_loop(0, N_PAD // 16, zero, 0)

    ones = jnp.ones((16,), jnp.float32)

    def count(i, carry):
        idx = didx[i >> 3, pl.ds((i & 7) * 16, 16)]
        plsc.addupdate_scatter(local, [idx >> 7, idx & 127], ones)
        return carry

    lax.fori_loop(0, N_PAD // 16, count, 0)

    pltpu.sync_copy(local, out_hbm.at[wid])


_ROWS_BLK = 2048


def _tc_pre(degcols, x, w0):
    def body(deg_ref, x_ref, w_ref, dis_ref, g_ref):
        deg = lax.dot_general(deg_ref[...], jnp.ones((NW, 1), jnp.float32),
                              (((0,), (0,)), ((), ())),
                              preferred_element_type=jnp.float32) + 1.0
        dis = lax.rsqrt(deg)
        dis_ref[...] = dis
        g_ref[...] = dis * jnp.dot(x_ref[...], w_ref[...],
                                   preferred_element_type=jnp.float32,
                                   precision=lax.Precision.DEFAULT)

    return pl.pallas_call(
        body,
        grid=(N_PAD // _ROWS_BLK,),
        in_specs=[
            pl.BlockSpec((NW, _ROWS_BLK), lambda i: (0, i)),
            pl.BlockSpec((_ROWS_BLK, DIM), lambda i: (i, 0)),
            pl.BlockSpec((DIM, DIM), lambda i: (0, 0)),
        ],
        out_specs=[
            pl.BlockSpec((_ROWS_BLK, 1), lambda i: (i, 0)),
            pl.BlockSpec((_ROWS_BLK, DIM), lambda i: (i, 0)),
        ],
        out_shape=[
            jax.ShapeDtypeStruct((N_PAD, 1), jnp.float32),
            jax.ShapeDtypeStruct((N_PAD, DIM), jnp.float32),
        ],
    )(degcols, x, w0)


def _tc_mid(s, dis, b, w_next):
    def body(s_ref, dis_ref, b_ref, w_ref, out_ref):
        t = dis_ref[...] * (s_ref[0] + s_ref[1]) + b_ref[...]
        h = jnp.maximum(t, 0.0)
        out_ref[...] = dis_ref[...] * jnp.dot(h, w_ref[...],
                                              preferred_element_type=jnp.float32,
                                              precision=lax.Precision.DEFAULT)

    return pl.pallas_call(
        body,
        grid=(N_PAD // _ROWS_BLK,),
        in_specs=[
            pl.BlockSpec((NUM_CORES, _ROWS_BLK, DIM), lambda i: (0, i, 0)),
            pl.BlockSpec((_ROWS_BLK, 1), lambda i: (i, 0)),
            pl.BlockSpec((1, DIM), lambda i: (0, 0)),
            pl.BlockSpec((DIM, DIM), lambda i: (0, 0)),
        ],
        out_specs=pl.BlockSpec((_ROWS_BLK, DIM), lambda i: (i, 0)),
        out_shape=jax.ShapeDtypeStruct((N_PAD, DIM), jnp.float32),
    )(s, dis, b, w_next)


def _tc_post(s, dis, b):
    def body(s_ref, dis_ref, b_ref, out_ref):
        out_ref[...] = dis_ref[...] * (s_ref[0] + s_ref[1]) + b_ref[...]

    blk = 1000
    return pl.pallas_call(
        body,
        grid=(N_NODES // blk,),
        in_specs=[
            pl.BlockSpec((NUM_CORES, blk, DIM), lambda i: (0, i, 0)),
            pl.BlockSpec((blk, 1), lambda i: (i, 0)),
            pl.BlockSpec((1, DIM), lambda i: (0, 0)),
        ],
        out_specs=pl.BlockSpec((blk, DIM), lambda i: (i, 0)),
        out_shape=jax.ShapeDtypeStruct((N_NODES, DIM), jnp.float32),
    )(s, dis, b)


def kernel(x, edge_index, Ws, bs):
    src = edge_index[0].astype(jnp.int32).reshape(NW, PHASES, HC, CHUNK)
    dst = edge_index[1].astype(jnp.int32).reshape(NW, PHASES, HC, CHUNK)
    dsti = edge_index[1].astype(jnp.int32)
    dstgrid = jnp.pad(dsti.reshape(NW, EDGES_PER_W),
                      ((0, 0), (0, N_PAD - EDGES_PER_W)),
                      constant_values=N_NODES).reshape(NW, N_PAD // DIM, DIM)
    zeros_d = jnp.zeros((N_PAD, DIM), jnp.float32)
    x_pad = jnp.pad(x, ((0, N_PAD - N_NODES), (0, 0)))

    degrows = _sc_degree(dstgrid).reshape(NW, N_PAD)
    dis, g = _tc_pre(degrows, x_pad, Ws[0])
    out = None
    for l in range(NUM_LAYERS):
        s = _sc_aggregate(g, src, dst, zeros_d)
        if l < NUM_LAYERS - 1:
            g = _tc_mid(s, dis, bs[l].reshape(1, DIM), Ws[l + 1])
        else:
            out = _tc_post(s, dis, bs[l].reshape(1, DIM))
    return out
